# baseline (speedup 1.0000x reference)
import functools

import jax
import jax.numpy as jnp
from jax import lax
from jax.experimental import pallas as pl
from jax.experimental.pallas import tpu as pltpu
from jax.experimental.pallas import tpu_sc as plsc

N_NODES = 100000
N_EDGES = 6400000
BATCH = 8

NC = 2
NS = 16
LANES = 16
NW = NC * NS
EDGES_PER_TILE = N_EDGES // NW
CHUNK = 1600
NCHUNKS = EDGES_PER_TILE // CHUNK
ROWS_PER_TILE = N_NODES // NS


NBUF = 4


def _sc_edge_kernel(stateT, srcs, tgts, weights, mask):
  mesh = plsc.VectorSubcoreMesh(core_axis_name="c", subcore_axis_name="s")

  buf_types = (
      [pltpu.VMEM((CHUNK,), jnp.int32) for _ in range(NBUF)]
      + [pltpu.VMEM((CHUNK,), jnp.int32) for _ in range(NBUF)]
      + [pltpu.VMEM((CHUNK,), jnp.float32) for _ in range(NBUF)]
      + [pltpu.VMEM((CHUNK,), jnp.float32) for _ in range(NBUF)]
      + [pltpu.VMEM((CHUNK, BATCH), jnp.float32) for _ in range(NBUF)]
      + [pltpu.SemaphoreType.DMA for _ in range(3 * NBUF)]
  )

  @functools.partial(
      pl.kernel,
      out_type=jax.ShapeDtypeStruct((NC, N_NODES, BATCH), jnp.float32),
      mesh=mesh,
      scratch_types=[
          pltpu.VMEM_SHARED((N_NODES, BATCH), jnp.float32),
      ] + buf_types,
      compiler_params=pltpu.CompilerParams(
          use_tc_tiling_on_sc=False, needs_layout_passes=False),
  )
  def kern(state_hbm, src_hbm, tgt_hbm, w_hbm, m_hbm, acc_hbm,
           acc_sh, *bufs):
    src_v = bufs[0:NBUF]
    tgt_v = bufs[NBUF:2 * NBUF]
    w_v = bufs[2 * NBUF:3 * NBUF]
    m_v = bufs[3 * NBUF:4 * NBUF]
    vals_v = bufs[4 * NBUF:5 * NBUF]
    sem_in = bufs[5 * NBUF:6 * NBUF]
    sem_g = bufs[6 * NBUF:7 * NBUF]
    sem_s = bufs[7 * NBUF:8 * NBUF]

    cid = lax.axis_index("c")
    sid = lax.axis_index("s")
    base = (cid * NS + sid) * EDGES_PER_TILE
    lane = lax.iota(jnp.int32, LANES)
    row_sel = lane // BATCH
    col_sel = lane % BATCH

    def issue_in(b, ic):
      off = base + ic * CHUNK
      pltpu.async_copy(src_hbm.at[pl.ds(off, CHUNK)], src_v[b], sem_in[b])
      pltpu.async_copy(tgt_hbm.at[pl.ds(off, CHUNK)], tgt_v[b], sem_in[b])
      pltpu.async_copy(w_hbm.at[pl.ds(off, CHUNK)], w_v[b], sem_in[b])
      pltpu.async_copy(m_hbm.at[pl.ds(off, CHUNK)], m_v[b], sem_in[b])

    def wait_in(b):
      pltpu.make_async_copy(src_hbm.at[pl.ds(0, CHUNK)], src_v[b],
                            sem_in[b]).wait()
      pltpu.make_async_copy(src_hbm.at[pl.ds(0, CHUNK)], tgt_v[b],
                            sem_in[b]).wait()
      pltpu.make_async_copy(w_hbm.at[pl.ds(0, CHUNK)], w_v[b],
                            sem_in[b]).wait()
      pltpu.make_async_copy(w_hbm.at[pl.ds(0, CHUNK)], m_v[b],
                            sem_in[b]).wait()

    def issue_gather(b):
      pltpu.async_copy(state_hbm.at[src_v[b]], vals_v[b], sem_g[b])

    def wait_gather(b):
      pltpu.make_async_copy(state_hbm.at[src_v[b]], vals_v[b], sem_g[b]).wait()

    def issue_scat(b):
      pltpu.async_copy(vals_v[b], acc_sh.at[tgt_v[b]], sem_s[b], add=True)

    def wait_scat(b):
      pltpu.make_async_copy(vals_v[b], acc_sh.at[tgt_v[b]], sem_s[b]).wait()

    def mask_mul(b):
      @plsc.parallel_loop(0, CHUNK, step=LANES, unroll=8)
      def _mask(j):
        w_v[b][pl.ds(j, LANES)] = (
            w_v[b][pl.ds(j, LANES)] * m_v[b][pl.ds(j, LANES)])

    def scale(b):
      @plsc.parallel_loop(0, CHUNK // 2, unroll=8)
      def _scale(j):
        row = row_sel + 2 * j
        wv = plsc.load_gather(w_v[b], [row])
        v = plsc.load_gather(vals_v[b], [row, col_sel])
        plsc.store_scatter(vals_v[b], [row, col_sel], v * wv)

    issue_in(0, 0)
    issue_in(1, 1)

    zeros16 = jnp.zeros((LANES,), jnp.float32)

    @plsc.parallel_loop(0, CHUNK // 2, unroll=8)
    def _zero(j):
      plsc.store_scatter(vals_v[0], [row_sel + 2 * j, col_sel], zeros16)

    r0 = sid * ROWS_PER_TILE
    for p in range(ROWS_PER_TILE // CHUNK):
      pltpu.sync_copy(vals_v[0], acc_sh.at[pl.ds(r0 + p * CHUNK, CHUNK)])
    rem = ROWS_PER_TILE % CHUNK
    if rem:
      pltpu.sync_copy(
          vals_v[0].at[pl.ds(0, rem)],
          acc_sh.at[pl.ds(r0 + (ROWS_PER_TILE // CHUNK) * CHUNK, rem)])
    plsc.subcore_barrier()

    wait_in(0)
    issue_gather(0)
    mask_mul(0)

    def chunk_body(j, u):
        b = u
        bn = (u + 1) % NBUF
        b2 = (u + 2) % NBUF

        @pl.when(j + 1 < NCHUNKS)
        def _prep_next():
          wait_in(bn)
          issue_gather(bn)
          mask_mul(bn)

        @pl.when(j + 2 < NCHUNKS)
        def _prefetch():
          @pl.when(j >= NBUF - 2)
          def _free_b2():
            wait_scat(b2)

          issue_in(b2, j + 2)

        wait_gather(b)
        scale(b)
        issue_scat(b)

    main_end = (NCHUNKS // NBUF) * NBUF

    @pl.loop(0, main_end, step=NBUF)
    def _pipe(j0):
      for u in range(NBUF):
        chunk_body(j0 + u, u)

    for u in range(NCHUNKS - main_end):
      chunk_body(main_end + u, u)

    for k in range(NBUF):
      wait_scat((NCHUNKS - NBUF + k) % NBUF)

    plsc.subcore_barrier()
    pltpu.sync_copy(acc_sh.at[pl.ds(r0, ROWS_PER_TILE)],
                    acc_hbm.at[cid, pl.ds(r0, ROWS_PER_TILE)])

  return kern(stateT, srcs, tgts, weights, mask)


def _tc_finish_kernel(state, injection, bias2d, decay2d, accT):
  def body(decay_ref, state_ref, inj_ref, bias_ref, acc0_ref, acc1_ref,
           out_ref):
    d = jax.nn.sigmoid(decay_ref[0, 0])
    raw = (d * state_ref[...] + inj_ref[...] + bias_ref[...]
           + acc0_ref[...] + acc1_ref[...])
    out_ref[...] = jnp.tanh(jnp.clip(raw, -5.0, 5.0))

  return pl.pallas_call(
      body,
      out_shape=jax.ShapeDtypeStruct((BATCH, N_NODES), jnp.float32),
      in_specs=[
          pl.BlockSpec(memory_space=pltpu.SMEM),
          pl.BlockSpec((BATCH, N_NODES), lambda: (0, 0)),
          pl.BlockSpec((BATCH, N_NODES), lambda: (0, 0)),
          pl.BlockSpec((1, N_NODES), lambda: (0, 0)),
          pl.BlockSpec((BATCH, N_NODES), lambda: (0, 0)),
          pl.BlockSpec((BATCH, N_NODES), lambda: (0, 0)),
      ],
      out_specs=pl.BlockSpec((BATCH, N_NODES), lambda: (0, 0)),
  )(decay2d, state, injection, bias2d, accT[0], accT[1])


def kernel(state, injection, bias, edge_weights, edge_mask, decay_logit,
           edge_sources, edge_targets):
  stateT = jnp.transpose(state)
  acc = _sc_edge_kernel(stateT, edge_sources, edge_targets,
                        edge_weights, edge_mask)
  accT = jnp.transpose(acc, (0, 2, 1))
  bias2d = bias[None, :]
  decay2d = jnp.reshape(decay_logit, (1, 1))
  return _tc_finish_kernel(state, injection, bias2d, decay2d, accT)


# speedup vs baseline: 1.0123x; 1.0123x over previous
import functools

import jax
import jax.numpy as jnp
from jax import lax
from jax.experimental import pallas as pl
from jax.experimental.pallas import tpu as pltpu
from jax.experimental.pallas import tpu_sc as plsc

N_NODES = 100000
N_EDGES = 6400000
BATCH = 8

NC = 2
NS = 16
LANES = 16
NW = NC * NS
EDGES_PER_TILE = N_EDGES // NW
CHUNK = 800
NCHUNKS = EDGES_PER_TILE // CHUNK
ROWS_PER_TILE = N_NODES // NS
S_LOCAL = 64000
SROWS_PER_TILE = S_LOCAL // NS


NBUF = 4


def _sc_edge_kernel(stateT, srcs, tgts, weights, mask):
  mesh = plsc.VectorSubcoreMesh(core_axis_name="c", subcore_axis_name="s")

  buf_types = (
      [pltpu.VMEM((CHUNK,), jnp.int32) for _ in range(NBUF)]
      + [pltpu.VMEM((CHUNK,), jnp.int32) for _ in range(NBUF)]
      + [pltpu.VMEM((CHUNK,), jnp.float32) for _ in range(NBUF)]
      + [pltpu.VMEM((CHUNK,), jnp.float32) for _ in range(NBUF)]
      + [pltpu.VMEM((CHUNK, BATCH), jnp.float32) for _ in range(NBUF)]
      + [pltpu.VMEM((CHUNK,), jnp.int32) for _ in range(NBUF)]
      + [pltpu.VMEM((CHUNK,), jnp.int32) for _ in range(NBUF)]
      + [pltpu.SemaphoreType.DMA for _ in range(4 * NBUF)]
  )

  @functools.partial(
      pl.kernel,
      out_type=jax.ShapeDtypeStruct((NC, N_NODES, BATCH), jnp.float32),
      mesh=mesh,
      scratch_types=[
          pltpu.VMEM_SHARED((N_NODES, BATCH), jnp.float32),
          pltpu.VMEM_SHARED((S_LOCAL, BATCH), jnp.float32),
      ] + buf_types,
      compiler_params=pltpu.CompilerParams(
          use_tc_tiling_on_sc=False, needs_layout_passes=False),
  )
  def kern(state_hbm, src_hbm, tgt_hbm, w_hbm, m_hbm, acc_hbm,
           acc_sh, state_sh, *bufs):
    src_v = bufs[0:NBUF]
    tgt_v = bufs[NBUF:2 * NBUF]
    w_v = bufs[2 * NBUF:3 * NBUF]
    m_v = bufs[3 * NBUF:4 * NBUF]
    vals_v = bufs[4 * NBUF:5 * NBUF]
    loc_v = bufs[5 * NBUF:6 * NBUF]
    far_v = bufs[6 * NBUF:7 * NBUF]
    sem_in = bufs[7 * NBUF:8 * NBUF]
    sem_g = bufs[8 * NBUF:9 * NBUF]
    sem_gf = bufs[9 * NBUF:10 * NBUF]
    sem_s = bufs[10 * NBUF:11 * NBUF]

    cid = lax.axis_index("c")
    sid = lax.axis_index("s")
    base = (cid * NS + sid) * EDGES_PER_TILE
    lane = lax.iota(jnp.int32, LANES)
    row_sel = lane // BATCH
    col_sel = lane % BATCH

    def issue_in(b, ic):
      off = base + ic * CHUNK
      pltpu.async_copy(src_hbm.at[pl.ds(off, CHUNK)], src_v[b], sem_in[b])
      pltpu.async_copy(tgt_hbm.at[pl.ds(off, CHUNK)], tgt_v[b], sem_in[b])
      pltpu.async_copy(w_hbm.at[pl.ds(off, CHUNK)], w_v[b], sem_in[b])
      pltpu.async_copy(m_hbm.at[pl.ds(off, CHUNK)], m_v[b], sem_in[b])

    def wait_in(b):
      pltpu.make_async_copy(src_hbm.at[pl.ds(0, CHUNK)], src_v[b],
                            sem_in[b]).wait()
      pltpu.make_async_copy(src_hbm.at[pl.ds(0, CHUNK)], tgt_v[b],
                            sem_in[b]).wait()
      pltpu.make_async_copy(w_hbm.at[pl.ds(0, CHUNK)], w_v[b],
                            sem_in[b]).wait()
      pltpu.make_async_copy(w_hbm.at[pl.ds(0, CHUNK)], m_v[b],
                            sem_in[b]).wait()

    def split_idx(b):
      @plsc.parallel_loop(0, CHUNK, step=LANES, unroll=8)
      def _split(j):
        s = src_v[b][pl.ds(j, LANES)]
        is_loc = s < S_LOCAL
        loc_v[b][pl.ds(j, LANES)] = jnp.where(is_loc, s, -1)
        far_v[b][pl.ds(j, LANES)] = jnp.where(is_loc, -1, s)

    def issue_gather(b):
      pltpu.async_copy(
          state_sh.at[plsc.Indices(loc_v[b], ignored_value=-1)],
          vals_v[b], sem_g[b])
      pltpu.async_copy(
          state_hbm.at[plsc.Indices(far_v[b], ignored_value=-1)],
          vals_v[b], sem_gf[b])

    def wait_gather(b):
      pltpu.make_async_copy(
          state_sh.at[plsc.Indices(loc_v[b], ignored_value=-1)],
          vals_v[b], sem_g[b]).wait()
      pltpu.make_async_copy(
          state_hbm.at[plsc.Indices(far_v[b], ignored_value=-1)],
          vals_v[b], sem_gf[b]).wait()

    def issue_scat(b):
      pltpu.async_copy(vals_v[b], acc_sh.at[tgt_v[b]], sem_s[b], add=True)

    def wait_scat(b):
      pltpu.make_async_copy(vals_v[b], acc_sh.at[tgt_v[b]], sem_s[b]).wait()

    def mask_mul(b):
      @plsc.parallel_loop(0, CHUNK, step=LANES, unroll=8)
      def _mask(j):
        w_v[b][pl.ds(j, LANES)] = (
            w_v[b][pl.ds(j, LANES)] * m_v[b][pl.ds(j, LANES)])

    def scale(b):
      @plsc.parallel_loop(0, CHUNK // 2, unroll=8)
      def _scale(j):
        row = row_sel + 2 * j
        wv = plsc.load_gather(w_v[b], [row])
        v = plsc.load_gather(vals_v[b], [row, col_sel])
        plsc.store_scatter(vals_v[b], [row, col_sel], v * wv)

    issue_in(0, 0)
    issue_in(1, 1)

    zeros16 = jnp.zeros((LANES,), jnp.float32)

    @plsc.parallel_loop(0, CHUNK // 2, unroll=8)
    def _zero(j):
      plsc.store_scatter(vals_v[0], [row_sel + 2 * j, col_sel], zeros16)

    rs = sid * SROWS_PER_TILE
    pltpu.sync_copy(state_hbm.at[pl.ds(rs, SROWS_PER_TILE)],
                    state_sh.at[pl.ds(rs, SROWS_PER_TILE)])
    r0 = sid * ROWS_PER_TILE
    for p in range(ROWS_PER_TILE // CHUNK):
      pltpu.sync_copy(vals_v[0], acc_sh.at[pl.ds(r0 + p * CHUNK, CHUNK)])
    rem = ROWS_PER_TILE % CHUNK
    if rem:
      pltpu.sync_copy(
          vals_v[0].at[pl.ds(0, rem)],
          acc_sh.at[pl.ds(r0 + (ROWS_PER_TILE // CHUNK) * CHUNK, rem)])
    plsc.subcore_barrier()

    wait_in(0)
    split_idx(0)
    issue_gather(0)
    mask_mul(0)

    def chunk_body(j, u):
        b = u
        bn = (u + 1) % NBUF
        b2 = (u + 2) % NBUF

        @pl.when(j + 1 < NCHUNKS)
        def _prep_next():
          wait_in(bn)
          split_idx(bn)
          issue_gather(bn)
          mask_mul(bn)

        @pl.when(j + 2 < NCHUNKS)
        def _prefetch():
          @pl.when(j >= NBUF - 2)
          def _free_b2():
            wait_scat(b2)

          issue_in(b2, j + 2)

        wait_gather(b)
        scale(b)
        issue_scat(b)

    main_end = (NCHUNKS // NBUF) * NBUF

    @pl.loop(0, main_end, step=NBUF)
    def _pipe(j0):
      for u in range(NBUF):
        chunk_body(j0 + u, u)

    for u in range(NCHUNKS - main_end):
      chunk_body(main_end + u, u)

    for k in range(NBUF):
      wait_scat((NCHUNKS - NBUF + k) % NBUF)

    plsc.subcore_barrier()
    pltpu.sync_copy(acc_sh.at[pl.ds(r0, ROWS_PER_TILE)],
                    acc_hbm.at[cid, pl.ds(r0, ROWS_PER_TILE)])

  return kern(stateT, srcs, tgts, weights, mask)


def _tc_finish_kernel(state, injection, bias2d, decay2d, accT):
  def body(decay_ref, state_ref, inj_ref, bias_ref, acc0_ref, acc1_ref,
           out_ref):
    d = jax.nn.sigmoid(decay_ref[0, 0])
    raw = (d * state_ref[...] + inj_ref[...] + bias_ref[...]
           + acc0_ref[...] + acc1_ref[...])
    out_ref[...] = jnp.tanh(jnp.clip(raw, -5.0, 5.0))

  return pl.pallas_call(
      body,
      out_shape=jax.ShapeDtypeStruct((BATCH, N_NODES), jnp.float32),
      in_specs=[
          pl.BlockSpec(memory_space=pltpu.SMEM),
          pl.BlockSpec((BATCH, N_NODES), lambda: (0, 0)),
          pl.BlockSpec((BATCH, N_NODES), lambda: (0, 0)),
          pl.BlockSpec((1, N_NODES), lambda: (0, 0)),
          pl.BlockSpec((BATCH, N_NODES), lambda: (0, 0)),
          pl.BlockSpec((BATCH, N_NODES), lambda: (0, 0)),
      ],
      out_specs=pl.BlockSpec((BATCH, N_NODES), lambda: (0, 0)),
  )(decay2d, state, injection, bias2d, accT[0], accT[1])


def kernel(state, injection, bias, edge_weights, edge_mask, decay_logit,
           edge_sources, edge_targets):
  stateT = jnp.transpose(state)
  acc = _sc_edge_kernel(stateT, edge_sources, edge_targets,
                        edge_weights, edge_mask)
  accT = jnp.transpose(acc, (0, 2, 1))
  bias2d = bias[None, :]
  decay2d = jnp.reshape(decay_logit, (1, 1))
  return _tc_finish_kernel(state, injection, bias2d, decay2d, accT)
